# Initial kernel scaffold; baseline (speedup 1.0000x reference)
import jax
import jax.numpy as jnp
from jax.experimental import pallas as pl


def kernel(x, codebook):
    raise NotImplementedError("write your pallas kernel here")


# speedup vs baseline: 86.9448x; 86.9448x over previous
import functools

import jax
import jax.numpy as jnp
from jax import lax
from jax.experimental import pallas as pl
from jax.experimental.pallas import tpu as pltpu
from jax.experimental.pallas import tpu_sc as plsc

_NUM_CODES = 8192
_DIM = 32
_TB = 512
_CCHUNK = 1024


def _argmin_body(x_ref, cb_ref, ids_ref):
    xb = x_ref[...]
    cb = cb_ref[...]
    x2 = jnp.sum(xb * xb, axis=1)
    c2 = jnp.sum(cb * cb, axis=1)

    best_l = jnp.full((_TB,), -jnp.inf, jnp.float32)
    best_i = jnp.zeros((_TB,), jnp.int32)
    for k in range(_NUM_CODES // _CCHUNK):
        cbk = cb[k * _CCHUNK:(k + 1) * _CCHUNK, :]
        c2k = c2[k * _CCHUNK:(k + 1) * _CCHUNK]
        xc = lax.dot_general(xb, cbk, (((1,), (1,)), ((), ())),
                             preferred_element_type=jnp.float32)
        d2 = jnp.maximum((x2[:, None] + c2k[None, :]) - 2.0 * xc, 0.0)
        l = -jnp.sqrt(d2)
        m = jnp.max(l, axis=1)
        ii = lax.broadcasted_iota(jnp.int32, (_TB, _CCHUNK), 1) + k * _CCHUNK
        idk = jnp.min(jnp.where(l == m[:, None], ii, jnp.int32(2**31 - 1)),
                      axis=1)
        upd = m > best_l
        best_i = jnp.where(upd, idk, best_i)
        best_l = jnp.where(upd, m, best_l)
    ids_ref[...] = best_i


def _argmin_ids(x2d, codebook):
    n_tok = x2d.shape[0]
    grid = n_tok // _TB
    return pl.pallas_call(
        _argmin_body,
        grid=(grid,),
        in_specs=[
            pl.BlockSpec((_TB, _DIM), lambda i: (i, 0)),
            pl.BlockSpec((_NUM_CODES, _DIM), lambda i: (0, 0)),
        ],
        out_specs=pl.BlockSpec((_TB,), lambda i: (i,)),
        out_shape=jax.ShapeDtypeStruct((n_tok,), jnp.int32),
    )(x2d, codebook)


def _sc_gather(codebook, ids):
    info = plsc.get_sparse_core_info()
    nc, ns = info.num_cores, info.num_subcores
    nw = nc * ns
    n_tok = ids.shape[0]
    b_per_w = n_tok // nw

    @functools.partial(
        pl.kernel,
        mesh=plsc.VectorSubcoreMesh(core_axis_name="c", subcore_axis_name="s"),
        compiler_params=pltpu.CompilerParams(use_tc_tiling_on_sc=False),
        out_type=jax.ShapeDtypeStruct((n_tok, _DIM), jnp.float32),
        scratch_types=[
            pltpu.VMEM((b_per_w,), jnp.int32),
            pltpu.VMEM((b_per_w, _DIM), jnp.float32),
            pltpu.SemaphoreType.DMA,
        ],
    )
    def gather_k(table_hbm, idx_hbm, out_hbm, idx_v, rows_v, sem):
        wid = lax.axis_index("s") * nc + lax.axis_index("c")
        base = wid * b_per_w
        pltpu.sync_copy(idx_hbm.at[pl.ds(base, b_per_w)], idx_v)
        pltpu.async_copy(table_hbm.at[idx_v], rows_v, sem).wait()
        pltpu.sync_copy(rows_v, out_hbm.at[pl.ds(base, b_per_w)])

    return gather_k(codebook, ids)


def kernel(x, codebook):
    b, t, d = x.shape
    x2d = x.reshape(b * t, d)
    ids = _argmin_ids(x2d, codebook)
    codes = _sc_gather(codebook, ids)
    return codes.reshape(b, t, d), ids.reshape(b, t, 1)


# speedup vs baseline: 92.4675x; 1.0635x over previous
import functools

import jax
import jax.numpy as jnp
from jax import lax
from jax.experimental import pallas as pl
from jax.experimental.pallas import tpu as pltpu
from jax.experimental.pallas import tpu_sc as plsc

_NUM_CODES = 8192
_DIM = 32
_TB = 512
_CCHUNK = 1024


def _argmin_body(x_ref, cb_ref, ids_ref):
    xb = x_ref[...]
    cb = cb_ref[...]
    x2 = jnp.sum(xb * xb, axis=1)
    c2 = jnp.sum(cb * cb, axis=1)
    cb2 = cb + cb
    ii = lax.broadcasted_iota(jnp.int32, (_TB, _CCHUNK), 1)

    best_s = jnp.full((_TB,), jnp.inf, jnp.float32)
    best_i = jnp.zeros((_TB,), jnp.int32)
    for k in range(_NUM_CODES // _CCHUNK):
        cbk = cb2[k * _CCHUNK:(k + 1) * _CCHUNK, :]
        c2k = c2[k * _CCHUNK:(k + 1) * _CCHUNK]
        xc2 = lax.dot_general(xb, cbk, (((1,), (1,)), ((), ())),
                              preferred_element_type=jnp.float32)
        d2 = jnp.maximum((x2[:, None] + c2k[None, :]) - xc2, 0.0)
        s = jnp.sqrt(d2)
        m = jnp.min(s, axis=1)
        idk = jnp.min(jnp.where(s == m[:, None], ii, jnp.int32(2**31 - 1)),
                      axis=1) + k * _CCHUNK
        upd = m < best_s
        best_i = jnp.where(upd, idk, best_i)
        best_s = jnp.where(upd, m, best_s)
    ids_ref[...] = best_i


def _argmin_ids(x2d, codebook):
    n_tok = x2d.shape[0]
    grid = n_tok // _TB
    return pl.pallas_call(
        _argmin_body,
        grid=(grid,),
        in_specs=[
            pl.BlockSpec((_TB, _DIM), lambda i: (i, 0)),
            pl.BlockSpec((_NUM_CODES, _DIM), lambda i: (0, 0)),
        ],
        out_specs=pl.BlockSpec((_TB,), lambda i: (i,)),
        out_shape=jax.ShapeDtypeStruct((n_tok,), jnp.int32),
    )(x2d, codebook)


def _sc_gather(codebook, ids):
    info = plsc.get_sparse_core_info()
    nc, ns = info.num_cores, info.num_subcores
    nw = nc * ns
    n_tok = ids.shape[0]
    b_per_w = n_tok // nw

    @functools.partial(
        pl.kernel,
        mesh=plsc.VectorSubcoreMesh(core_axis_name="c", subcore_axis_name="s"),
        compiler_params=pltpu.CompilerParams(use_tc_tiling_on_sc=False),
        out_type=jax.ShapeDtypeStruct((n_tok, _DIM), jnp.float32),
        scratch_types=[
            pltpu.VMEM((b_per_w,), jnp.int32),
            pltpu.VMEM((b_per_w, _DIM), jnp.float32),
            pltpu.SemaphoreType.DMA,
        ],
    )
    def gather_k(table_hbm, idx_hbm, out_hbm, idx_v, rows_v, sem):
        wid = lax.axis_index("s") * nc + lax.axis_index("c")
        base = wid * b_per_w
        pltpu.sync_copy(idx_hbm.at[pl.ds(base, b_per_w)], idx_v)
        pltpu.async_copy(table_hbm.at[idx_v], rows_v, sem).wait()
        pltpu.sync_copy(rows_v, out_hbm.at[pl.ds(base, b_per_w)])

    return gather_k(codebook, ids)


def kernel(x, codebook):
    b, t, d = x.shape
    x2d = x.reshape(b * t, d)
    ids = _argmin_ids(x2d, codebook)
    codes = _sc_gather(codebook, ids)
    return codes.reshape(b, t, d), ids.reshape(b, t, 1)
